# baseline (speedup 1.0000x reference)
import functools

import jax
import jax.numpy as jnp
from jax import lax
from jax.experimental import pallas as pl
from jax.experimental.pallas import tpu as pltpu
from jax.experimental.pallas import tpu_sc as plsc

_NE = 10000
_NEDGE = 320000
_NR = 24
_NB = 8
_DE = 128
_HID = 2048

_NC = 2
_NS = 16
_NW = _NC * _NS

_EPAD = 327680
_EPT = _EPAD // _NW
_EPT_H = _EPAD // _NS
_OUT_ROWS = 10240
_RPT = _OUT_ROWS // _NS
_NKEY = 241664
_KPT = _NKEY // _NS
_NSINK = 64
_CB = 2048
_SB = 128
_ZB = _KPT


def _w_body(comp_ref, bases_ref, w_ref):
    b = bases_ref[...].reshape(_NB, -1)
    w = jnp.dot(comp_ref[...], b, preferred_element_type=jnp.float32)
    w_ref[...] = w.reshape(_NR, w_ref.shape[1], _DE)


def _compute_w(comp, bases):
    bi = 1000
    return pl.pallas_call(
        _w_body,
        grid=(_NE // bi,),
        in_specs=[
            pl.BlockSpec((_NR, _NB), lambda i: (0, 0)),
            pl.BlockSpec((_NB, bi, _DE), lambda i: (0, i, 0)),
        ],
        out_specs=pl.BlockSpec((_NR, bi, _DE), lambda i: (0, i, 0)),
        out_shape=jax.ShapeDtypeStruct((_NR, _NE, _DE), jnp.float32),
    )(comp, bases)


def _sc_hist_kernel(dst_h, et_h, scale_h, hist, zb, dstv, etv, keyc, onesb, hsem):
    c = lax.axis_index("c")
    s = lax.axis_index("s")

    def _zb_zero(i, _):
        zb[pl.ds(i * 16, 16)] = jnp.zeros((16,), jnp.float32)
        return 0
    lax.fori_loop(0, _ZB // 16, _zb_zero, 0)
    pltpu.sync_copy(zb, hist.at[pl.ds(s * _KPT, _KPT)])

    def _ones(i, _):
        onesb[pl.ds(i * 16, 16)] = jnp.full((16,), 1.0, jnp.float32)
        return 0
    lax.fori_loop(0, _SB // 16, _ones, 0)

    plsc.subcore_barrier()

    hbase = s * _EPT_H

    def _hchunk(ci, _):
        off = hbase + ci * _CB
        pltpu.sync_copy(dst_h.at[pl.ds(off, _CB)], dstv)
        pltpu.sync_copy(et_h.at[pl.ds(off, _CB)], etv)

        def _hidx(i, _):
            o = pl.multiple_of(i * 16, 16)
            r = i // 8
            q = pl.multiple_of((i % 8) * 16, 16)
            keyc[r, pl.ds(q, 16)] = dstv[pl.ds(o, 16)] * _NR + etv[pl.ds(o, 16)]
            return 0
        lax.fori_loop(0, _CB // 16, _hidx, 0)

        descs = [pltpu.async_copy(onesb, hist.at[keyc.at[j]], hsem, add=True)
                 for j in range(_CB // _SB)]
        for d in descs:
            d.wait()
        return 0
    lax.fori_loop(0, _EPT_H // _CB, _hchunk, 0)

    plsc.subcore_barrier()

    pltpu.sync_copy(hist.at[pl.ds(s * _KPT, _KPT)], zb)

    def _conv(i, _):
        v = zb[pl.ds(i * 16, 16)]
        zb[pl.ds(i * 16, 16)] = 1.0 / jnp.maximum(v, 1.0)
        return 0
    lax.fori_loop(0, _KPT // 16, _conv, 0)

    @pl.when(c == 0)
    def _():
        pltpu.sync_copy(zb, scale_h.at[pl.ds(s * _KPT, _KPT)])


def _sc_hist(dst_p, et_p):
    mesh = plsc.VectorSubcoreMesh(core_axis_name="c", subcore_axis_name="s")
    f = pl.kernel(
        _sc_hist_kernel,
        out_type=jax.ShapeDtypeStruct((_NKEY,), jnp.float32),
        mesh=mesh,
        scratch_types=[
            pltpu.VMEM_SHARED((_NKEY,), jnp.float32),
            pltpu.VMEM((_ZB,), jnp.float32),
            pltpu.VMEM((_CB,), jnp.int32),
            pltpu.VMEM((_CB,), jnp.int32),
            pltpu.VMEM((_CB // _SB, _SB), jnp.int32),
            pltpu.VMEM((_SB,), jnp.float32),
            pltpu.SemaphoreType.DMA,
        ],
    )
    return f(dst_p, et_p)


_NBATCH = _CB // _SB
_CH0 = 8
_CH1 = 2


def _sc_main_kernel(src_h, dst_h, et_h, w_h, scale_h, out_h,
                    acc, srcv, dstv, etv, widxc, keyc, dstc,
                    scaleb0, scaleb1, rowb0, rowb1,
                    gsem0, gsem1, ksem0, ksem1, ssem0, ssem1):
    c = lax.axis_index("c")
    s = lax.axis_index("s")
    rowb = (rowb0, rowb1)
    scaleb = (scaleb0, scaleb1)
    gsem = (gsem0, gsem1)
    ksem = (ksem0, ksem1)
    ssem = (ssem0, ssem1)

    def _rowb_zero(i, _):
        rowb0[i // 8, pl.ds(pl.multiple_of((i % 8) * 16, 16), 16)] = (
            jnp.zeros((16,), jnp.float32))
        return 0
    lax.fori_loop(0, _SB * 8, _rowb_zero, 0)

    def _acc_zero(j, _):
        pltpu.sync_copy(rowb0, acc.at[pl.ds(s * _RPT + j * _SB, _SB)])
        return 0
    lax.fori_loop(0, _RPT // _SB, _acc_zero, 0)

    plsc.subcore_barrier()

    base = s * (_NC * _EPT) + jnp.where(c == 0, 0, _CH0 * _CB)
    nchunks = jnp.where(c == 0, _CH0, _CH1)

    def _chunk(ci, _):
        off = base + ci * _CB
        pltpu.sync_copy(src_h.at[pl.ds(off, _CB)], srcv)
        pltpu.sync_copy(dst_h.at[pl.ds(off, _CB)], dstv)
        pltpu.sync_copy(et_h.at[pl.ds(off, _CB)], etv)

        def _idx(i, _):
            o = pl.multiple_of(i * 16, 16)
            sv = srcv[pl.ds(o, 16)]
            tv = etv[pl.ds(o, 16)]
            dv = dstv[pl.ds(o, 16)]
            r = i // 8
            q = pl.multiple_of((i % 8) * 16, 16)
            widxc[r, pl.ds(q, 16)] = tv * _NE + sv
            keyc[r, pl.ds(q, 16)] = dv * _NR + tv
            dstc[r, pl.ds(q, 16)] = dv
            return 0
        lax.fori_loop(0, _CB // 16, _idx, 0)

        gd = [None, None]
        kd = [None, None]
        sd = [None, None]

        def _issue(k):
            b = k % 2
            if sd[b] is not None:
                sd[b].wait()
            gd[b] = pltpu.async_copy(w_h.at[widxc.at[k]], rowb[b], gsem[b])
            kd[b] = pltpu.async_copy(scale_h.at[keyc.at[k]], scaleb[b], ksem[b])

        _issue(0)
        for j in range(_NBATCH):
            if j + 1 < _NBATCH:
                _issue(j + 1)
            b = j % 2
            gd[b].wait()
            kd[b].wait()

            def _scale(g, _):
                sv = scaleb[b][pl.ds(pl.multiple_of(g * 16, 16), 16)]
                for l in range(16):
                    sc = sv[l]
                    r = g * 16 + l
                    for q in range(_DE // 16):
                        rowb[b][r, pl.ds(q * 16, 16)] = (
                            rowb[b][r, pl.ds(q * 16, 16)] * sc)
                return 0
            lax.fori_loop(0, _SB // 16, _scale, 0)

            sd[b] = pltpu.async_copy(rowb[b], acc.at[dstc.at[j]], ssem[b],
                                     add=True)
        sd[0].wait()
        sd[1].wait()
        return 0
    lax.fori_loop(0, nchunks, _chunk, 0)

    plsc.subcore_barrier()

    row0 = s * _RPT

    def _dump(j, _):
        pltpu.sync_copy(acc.at[pl.ds(row0 + j * _SB, _SB)], rowb0)
        pltpu.sync_copy(rowb0, out_h.at[pl.ds(c * _OUT_ROWS + row0 + j * _SB, _SB)])
        return 0
    lax.fori_loop(0, _RPT // _SB, _dump, 0)


def _sc_main(src_p, dst_p, et_p, w_flat, scales):
    mesh = plsc.VectorSubcoreMesh(core_axis_name="c", subcore_axis_name="s")
    f = pl.kernel(
        _sc_main_kernel,
        out_type=jax.ShapeDtypeStruct((_NC * _OUT_ROWS, _DE), jnp.float32),
        mesh=mesh,
        scratch_types=[
            pltpu.VMEM_SHARED((_OUT_ROWS, _DE), jnp.float32),
            pltpu.VMEM((_CB,), jnp.int32),
            pltpu.VMEM((_CB,), jnp.int32),
            pltpu.VMEM((_CB,), jnp.int32),
            pltpu.VMEM((_NBATCH, _SB), jnp.int32),
            pltpu.VMEM((_NBATCH, _SB), jnp.int32),
            pltpu.VMEM((_NBATCH, _SB), jnp.int32),
            pltpu.VMEM((_SB,), jnp.float32),
            pltpu.VMEM((_SB,), jnp.float32),
            pltpu.VMEM((_SB, _DE), jnp.float32),
            pltpu.VMEM((_SB, _DE), jnp.float32),
            pltpu.SemaphoreType.DMA,
            pltpu.SemaphoreType.DMA,
            pltpu.SemaphoreType.DMA,
            pltpu.SemaphoreType.DMA,
            pltpu.SemaphoreType.DMA,
            pltpu.SemaphoreType.DMA,
        ],
    )
    return f(src_p, dst_p, et_p, w_flat, scales)


_FR = 160


def _sc_finish_kernel(pp, rootp, bias_h, ids_h, out_all, ent,
                      b0, b1, b2, biasv, idsb, ids1b, e0, e1, e2):
    c = lax.axis_index("c")
    s = lax.axis_index("s")
    gid = s * _NC + c
    pltpu.sync_copy(bias_h, biasv)

    base = gid * (_OUT_ROWS // _NW)

    def _rchunk(j, _):
        r0 = base + j * _FR
        pltpu.sync_copy(pp.at[pl.ds(r0, _FR)], b0)
        pltpu.sync_copy(pp.at[pl.ds(_OUT_ROWS + r0, _FR)], b1)
        pltpu.sync_copy(rootp.at[pl.ds(r0, _FR)], b2)

        def _add(i, _):
            r = i // 8
            q = pl.multiple_of((i % 8) * 16, 16)
            b0[r, pl.ds(q, 16)] = (b0[r, pl.ds(q, 16)] + b1[r, pl.ds(q, 16)]
                                   + b2[r, pl.ds(q, 16)] + biasv[pl.ds(q, 16)])
            return 0
        lax.fori_loop(0, _FR * 8, _add, 0)
        pltpu.sync_copy(b0, out_all.at[pl.ds(r0, _FR)])
        return 0
    lax.fori_loop(0, (_OUT_ROWS // _NW) // _FR, _rchunk, 0)

    pltpu.sync_copy(ids_h.at[pl.ds(gid * 32, 32)], idsb)

    def _sh(i, _):
        ids1b[pl.ds(i * 16, 16)] = idsb[pl.ds(i * 16, 16)] + _OUT_ROWS
        return 0
    lax.fori_loop(0, 2, _sh, 0)

    pltpu.sync_copy(pp.at[idsb], e0)
    pltpu.sync_copy(pp.at[ids1b], e1)
    pltpu.sync_copy(rootp.at[idsb], e2)

    def _eadd(i, _):
        r = i // 8
        q = pl.multiple_of((i % 8) * 16, 16)
        e0[r, pl.ds(q, 16)] = (e0[r, pl.ds(q, 16)] + e1[r, pl.ds(q, 16)]
                               + e2[r, pl.ds(q, 16)] + biasv[pl.ds(q, 16)])
        return 0
    lax.fori_loop(0, 32 * 8, _eadd, 0)
    pltpu.sync_copy(e0, ent.at[pl.ds(gid * 32, 32)])


def _sc_finish(parts, root_p, bias, ids):
    mesh = plsc.VectorSubcoreMesh(core_axis_name="c", subcore_axis_name="s")
    f = pl.kernel(
        _sc_finish_kernel,
        out_type=(jax.ShapeDtypeStruct((_OUT_ROWS, _DE), jnp.float32),
                  jax.ShapeDtypeStruct((_NW * 32, _DE), jnp.float32)),
        mesh=mesh,
        scratch_types=[
            pltpu.VMEM((_FR, _DE), jnp.float32),
            pltpu.VMEM((_FR, _DE), jnp.float32),
            pltpu.VMEM((_FR, _DE), jnp.float32),
            pltpu.VMEM((_DE,), jnp.float32),
            pltpu.VMEM((32,), jnp.int32),
            pltpu.VMEM((32,), jnp.int32),
            pltpu.VMEM((32, _DE), jnp.float32),
            pltpu.VMEM((32, _DE), jnp.float32),
            pltpu.VMEM((32, _DE), jnp.float32),
        ],
    )
    return f(parts, root_p, bias, ids)


def _tail_body(ent_ref, w1_ref, b1_ref, w2_ref, b2_ref, wp_ref, bp_ref, out_ref):
    ent = ent_ref[...]
    h = jnp.maximum(jnp.dot(ent, w1_ref[...], preferred_element_type=jnp.float32)
                    + b1_ref[...], 0.0)
    h = (jnp.dot(h, w2_ref[...], preferred_element_type=jnp.float32)
         + b2_ref[...] + ent)
    out_ref[...] = (jnp.dot(h, wp_ref[...], preferred_element_type=jnp.float32)
                    + bp_ref[...])


def _tc_tail(ent, w1, b1, w2, b2, wp, bp):
    return pl.pallas_call(
        _tail_body,
        out_shape=jax.ShapeDtypeStruct((ent.shape[0], _HID), jnp.float32),
    )(ent, w1, b1.reshape(1, -1), w2, b2.reshape(1, -1), wp, bp.reshape(1, -1))


def kernel(entity_ids, edge_index, edge_type, bases, comp, root, bias,
           w1, b1, w2, b2, wp, bp):
    src = edge_index[0].astype(jnp.int32)
    dst = edge_index[1].astype(jnp.int32)
    et = edge_type.astype(jnp.int32)
    npad = _EPAD - _NEDGE
    src_p = jnp.concatenate([src, jnp.zeros((npad,), jnp.int32)])
    dst_p = jnp.concatenate(
        [dst, _NE + (jnp.arange(npad, dtype=jnp.int32) % _NSINK)])
    et_p = jnp.concatenate([et, jnp.zeros((npad,), jnp.int32)])

    scales = _sc_hist(dst_p, et_p)
    w = _compute_w(comp, bases)
    w_flat = w.reshape(_NR * _NE, _DE)

    parts = _sc_main(src_p, dst_p, et_p, w_flat, scales)

    root_p = jnp.concatenate(
        [root, jnp.zeros((_OUT_ROWS - _NE, _DE), jnp.float32)])
    ids = entity_ids.reshape(-1).astype(jnp.int32)
    out_all_p, ent = _sc_finish(parts, root_p, bias, ids)

    entity_embeds_all = out_all_p[:_NE]
    emb = _tc_tail(ent, w1, b1, w2, b2, wp, bp)
    entity_embeds = emb.reshape(entity_ids.shape[0], entity_ids.shape[1], _HID)
    return (entity_embeds, entity_embeds_all)


# speedup vs baseline: 1.0067x; 1.0067x over previous
import functools

import jax
import jax.numpy as jnp
from jax import lax
from jax.experimental import pallas as pl
from jax.experimental.pallas import tpu as pltpu
from jax.experimental.pallas import tpu_sc as plsc

_NE = 10000
_NEDGE = 320000
_NR = 24
_NB = 8
_DE = 128
_HID = 2048

_NC = 2
_NS = 16
_NW = _NC * _NS

_EPAD = 327680
_EPT = _EPAD // _NW
_EPT_H = _EPAD // _NS
_OUT_ROWS = 10240
_RPT = _OUT_ROWS // _NS
_NKEY = 241664
_KPT = _NKEY // _NS
_NSINK = 64
_CB = 2048
_SB = 128
_ZB = _KPT


def _w_body(comp_ref, bases_ref, w_ref):
    b = bases_ref[...].reshape(_NB, -1)
    w = jnp.dot(comp_ref[...], b, preferred_element_type=jnp.float32)
    w_ref[...] = w.reshape(_NR, w_ref.shape[1], _DE)


def _compute_w(comp, bases):
    bi = 1000
    return pl.pallas_call(
        _w_body,
        grid=(_NE // bi,),
        in_specs=[
            pl.BlockSpec((_NR, _NB), lambda i: (0, 0)),
            pl.BlockSpec((_NB, bi, _DE), lambda i: (0, i, 0)),
        ],
        out_specs=pl.BlockSpec((_NR, bi, _DE), lambda i: (0, i, 0)),
        out_shape=jax.ShapeDtypeStruct((_NR, _NE, _DE), jnp.float32),
    )(comp, bases)


def _sc_hist_kernel(dst_h, et_h, scale_h, hist, zb, dstv, etv, keyc, onesb, hsem):
    c = lax.axis_index("c")
    s = lax.axis_index("s")

    def _zb_zero(i, _):
        zb[pl.ds(i * 16, 16)] = jnp.zeros((16,), jnp.float32)
        return 0
    lax.fori_loop(0, _ZB // 16, _zb_zero, 0)
    pltpu.sync_copy(zb, hist.at[pl.ds(s * _KPT, _KPT)])

    def _ones(i, _):
        onesb[pl.ds(i * 16, 16)] = jnp.full((16,), 1.0, jnp.float32)
        return 0
    lax.fori_loop(0, _SB // 16, _ones, 0)

    plsc.subcore_barrier()

    hbase = s * _EPT_H

    def _hchunk(ci, _):
        off = hbase + ci * _CB
        pltpu.sync_copy(dst_h.at[pl.ds(off, _CB)], dstv)
        pltpu.sync_copy(et_h.at[pl.ds(off, _CB)], etv)

        def _hidx(i, _):
            o = pl.multiple_of(i * 16, 16)
            r = i // 8
            q = pl.multiple_of((i % 8) * 16, 16)
            keyc[r, pl.ds(q, 16)] = dstv[pl.ds(o, 16)] * _NR + etv[pl.ds(o, 16)]
            return 0
        lax.fori_loop(0, _CB // 16, _hidx, 0)

        descs = [pltpu.async_copy(onesb, hist.at[keyc.at[j]], hsem, add=True)
                 for j in range(_CB // _SB)]
        for d in descs:
            d.wait()
        return 0
    lax.fori_loop(0, _EPT_H // _CB, _hchunk, 0)

    plsc.subcore_barrier()

    pltpu.sync_copy(hist.at[pl.ds(s * _KPT, _KPT)], zb)

    def _conv(i, _):
        v = zb[pl.ds(i * 16, 16)]
        zb[pl.ds(i * 16, 16)] = 1.0 / jnp.maximum(v, 1.0)
        return 0
    lax.fori_loop(0, _KPT // 16, _conv, 0)

    @pl.when(c == 0)
    def _():
        pltpu.sync_copy(zb, scale_h.at[pl.ds(s * _KPT, _KPT)])


def _sc_hist(dst_p, et_p):
    mesh = plsc.VectorSubcoreMesh(core_axis_name="c", subcore_axis_name="s")
    f = pl.kernel(
        _sc_hist_kernel,
        out_type=jax.ShapeDtypeStruct((_NKEY,), jnp.float32),
        mesh=mesh,
        scratch_types=[
            pltpu.VMEM_SHARED((_NKEY,), jnp.float32),
            pltpu.VMEM((_ZB,), jnp.float32),
            pltpu.VMEM((_CB,), jnp.int32),
            pltpu.VMEM((_CB,), jnp.int32),
            pltpu.VMEM((_CB // _SB, _SB), jnp.int32),
            pltpu.VMEM((_SB,), jnp.float32),
            pltpu.SemaphoreType.DMA,
        ],
    )
    return f(dst_p, et_p)


_NBATCH = _CB // _SB
_CH0 = 7
_CH1 = 3


def _sc_main_kernel(src_h, dst_h, et_h, w_h, scale_h, out_h,
                    acc, srcv, dstv, etv, widxc, keyc, dstc,
                    scaleb0, scaleb1, rowb0, rowb1,
                    gsem0, gsem1, ksem0, ksem1, ssem0, ssem1):
    c = lax.axis_index("c")
    s = lax.axis_index("s")
    rowb = (rowb0, rowb1)
    scaleb = (scaleb0, scaleb1)
    gsem = (gsem0, gsem1)
    ksem = (ksem0, ksem1)
    ssem = (ssem0, ssem1)

    def _rowb_zero(i, _):
        rowb0[i // 8, pl.ds(pl.multiple_of((i % 8) * 16, 16), 16)] = (
            jnp.zeros((16,), jnp.float32))
        return 0
    lax.fori_loop(0, _SB * 8, _rowb_zero, 0)

    def _acc_zero(j, _):
        pltpu.sync_copy(rowb0, acc.at[pl.ds(s * _RPT + j * _SB, _SB)])
        return 0
    lax.fori_loop(0, _RPT // _SB, _acc_zero, 0)

    plsc.subcore_barrier()

    base = s * (_NC * _EPT) + jnp.where(c == 0, 0, _CH0 * _CB)
    nchunks = jnp.where(c == 0, _CH0, _CH1)

    def _chunk(ci, _):
        off = base + ci * _CB
        pltpu.sync_copy(src_h.at[pl.ds(off, _CB)], srcv)
        pltpu.sync_copy(dst_h.at[pl.ds(off, _CB)], dstv)
        pltpu.sync_copy(et_h.at[pl.ds(off, _CB)], etv)

        def _idx(i, _):
            o = pl.multiple_of(i * 16, 16)
            sv = srcv[pl.ds(o, 16)]
            tv = etv[pl.ds(o, 16)]
            dv = dstv[pl.ds(o, 16)]
            r = i // 8
            q = pl.multiple_of((i % 8) * 16, 16)
            widxc[r, pl.ds(q, 16)] = tv * _NE + sv
            keyc[r, pl.ds(q, 16)] = dv * _NR + tv
            dstc[r, pl.ds(q, 16)] = dv
            return 0
        lax.fori_loop(0, _CB // 16, _idx, 0)

        gd = [None, None]
        kd = [None, None]
        sd = [None, None]

        def _issue(k):
            b = k % 2
            if sd[b] is not None:
                sd[b].wait()
            gd[b] = pltpu.async_copy(w_h.at[widxc.at[k]], rowb[b], gsem[b])
            kd[b] = pltpu.async_copy(scale_h.at[keyc.at[k]], scaleb[b], ksem[b])

        _issue(0)
        for j in range(_NBATCH):
            if j + 1 < _NBATCH:
                _issue(j + 1)
            b = j % 2
            gd[b].wait()
            kd[b].wait()

            def _scale(g, _):
                sv = scaleb[b][pl.ds(pl.multiple_of(g * 16, 16), 16)]
                for l in range(16):
                    sc = sv[l]
                    r = g * 16 + l
                    for q in range(_DE // 16):
                        rowb[b][r, pl.ds(q * 16, 16)] = (
                            rowb[b][r, pl.ds(q * 16, 16)] * sc)
                return 0
            lax.fori_loop(0, _SB // 16, _scale, 0)

            sd[b] = pltpu.async_copy(rowb[b], acc.at[dstc.at[j]], ssem[b],
                                     add=True)
        sd[0].wait()
        sd[1].wait()
        return 0
    lax.fori_loop(0, nchunks, _chunk, 0)

    plsc.subcore_barrier()

    row0 = s * _RPT

    def _dump(j, _):
        pltpu.sync_copy(acc.at[pl.ds(row0 + j * _SB, _SB)], rowb0)
        pltpu.sync_copy(rowb0, out_h.at[pl.ds(c * _OUT_ROWS + row0 + j * _SB, _SB)])
        return 0
    lax.fori_loop(0, _RPT // _SB, _dump, 0)


def _sc_main(src_p, dst_p, et_p, w_flat, scales):
    mesh = plsc.VectorSubcoreMesh(core_axis_name="c", subcore_axis_name="s")
    f = pl.kernel(
        _sc_main_kernel,
        out_type=jax.ShapeDtypeStruct((_NC * _OUT_ROWS, _DE), jnp.float32),
        mesh=mesh,
        scratch_types=[
            pltpu.VMEM_SHARED((_OUT_ROWS, _DE), jnp.float32),
            pltpu.VMEM((_CB,), jnp.int32),
            pltpu.VMEM((_CB,), jnp.int32),
            pltpu.VMEM((_CB,), jnp.int32),
            pltpu.VMEM((_NBATCH, _SB), jnp.int32),
            pltpu.VMEM((_NBATCH, _SB), jnp.int32),
            pltpu.VMEM((_NBATCH, _SB), jnp.int32),
            pltpu.VMEM((_SB,), jnp.float32),
            pltpu.VMEM((_SB,), jnp.float32),
            pltpu.VMEM((_SB, _DE), jnp.float32),
            pltpu.VMEM((_SB, _DE), jnp.float32),
            pltpu.SemaphoreType.DMA,
            pltpu.SemaphoreType.DMA,
            pltpu.SemaphoreType.DMA,
            pltpu.SemaphoreType.DMA,
            pltpu.SemaphoreType.DMA,
            pltpu.SemaphoreType.DMA,
        ],
    )
    return f(src_p, dst_p, et_p, w_flat, scales)


_FR = 160


def _sc_sum_kernel(pp, rootp, bias_h, out_all,
                   b0, b1, b2, biasv):
    c = lax.axis_index("c")
    s = lax.axis_index("s")
    gid = s * _NC + c
    pltpu.sync_copy(bias_h, biasv)

    base = gid * (_OUT_ROWS // _NW)

    def _rchunk(j, _):
        r0 = base + j * _FR
        pltpu.sync_copy(pp.at[pl.ds(r0, _FR)], b0)
        pltpu.sync_copy(pp.at[pl.ds(_OUT_ROWS + r0, _FR)], b1)
        pltpu.sync_copy(rootp.at[pl.ds(r0, _FR)], b2)

        def _add(i, _):
            r = i // 8
            q = pl.multiple_of((i % 8) * 16, 16)
            b0[r, pl.ds(q, 16)] = (b0[r, pl.ds(q, 16)] + b1[r, pl.ds(q, 16)]
                                   + b2[r, pl.ds(q, 16)] + biasv[pl.ds(q, 16)])
            return 0
        lax.fori_loop(0, _FR * 8, _add, 0)
        pltpu.sync_copy(b0, out_all.at[pl.ds(r0, _FR)])
        return 0
    lax.fori_loop(0, (_OUT_ROWS // _NW) // _FR, _rchunk, 0)


def _sc_sum(parts, root_p, bias):
    mesh = plsc.VectorSubcoreMesh(core_axis_name="c", subcore_axis_name="s")
    f = pl.kernel(
        _sc_sum_kernel,
        out_type=jax.ShapeDtypeStruct((_OUT_ROWS, _DE), jnp.float32),
        mesh=mesh,
        scratch_types=[
            pltpu.VMEM((_FR, _DE), jnp.float32),
            pltpu.VMEM((_FR, _DE), jnp.float32),
            pltpu.VMEM((_FR, _DE), jnp.float32),
            pltpu.VMEM((_DE,), jnp.float32),
        ],
    )
    return f(parts, root_p, bias)


def _sc_ent_kernel(pp, rootp, bias_h, ids_h, ent,
                   biasv, idsb, ids1b, e0, e1, e2):
    c = lax.axis_index("c")
    s = lax.axis_index("s")
    gid = s * _NC + c
    pltpu.sync_copy(bias_h, biasv)

    pltpu.sync_copy(ids_h.at[pl.ds(gid * 32, 32)], idsb)

    def _sh(i, _):
        ids1b[pl.ds(i * 16, 16)] = idsb[pl.ds(i * 16, 16)] + _OUT_ROWS
        return 0
    lax.fori_loop(0, 2, _sh, 0)

    pltpu.sync_copy(pp.at[idsb], e0)
    pltpu.sync_copy(pp.at[ids1b], e1)
    pltpu.sync_copy(rootp.at[idsb], e2)

    def _eadd(i, _):
        r = i // 8
        q = pl.multiple_of((i % 8) * 16, 16)
        e0[r, pl.ds(q, 16)] = (e0[r, pl.ds(q, 16)] + e1[r, pl.ds(q, 16)]
                               + e2[r, pl.ds(q, 16)] + biasv[pl.ds(q, 16)])
        return 0
    lax.fori_loop(0, 32 * 8, _eadd, 0)
    pltpu.sync_copy(e0, ent.at[pl.ds(gid * 32, 32)])


def _sc_ent(parts, root_p, bias, ids):
    mesh = plsc.VectorSubcoreMesh(core_axis_name="c", subcore_axis_name="s")
    f = pl.kernel(
        _sc_ent_kernel,
        out_type=jax.ShapeDtypeStruct((_NW * 32, _DE), jnp.float32),
        mesh=mesh,
        scratch_types=[
            pltpu.VMEM((_DE,), jnp.float32),
            pltpu.VMEM((32,), jnp.int32),
            pltpu.VMEM((32,), jnp.int32),
            pltpu.VMEM((32, _DE), jnp.float32),
            pltpu.VMEM((32, _DE), jnp.float32),
            pltpu.VMEM((32, _DE), jnp.float32),
        ],
    )
    return f(parts, root_p, bias, ids)


def _tail_body(ent_ref, w1_ref, b1_ref, w2_ref, b2_ref, wp_ref, bp_ref, out_ref):
    ent = ent_ref[...]
    h = jnp.maximum(jnp.dot(ent, w1_ref[...], preferred_element_type=jnp.float32)
                    + b1_ref[...], 0.0)
    h = (jnp.dot(h, w2_ref[...], preferred_element_type=jnp.float32)
         + b2_ref[...] + ent)
    out_ref[...] = (jnp.dot(h, wp_ref[...], preferred_element_type=jnp.float32)
                    + bp_ref[...])


def _tc_tail(ent, w1, b1, w2, b2, wp, bp):
    return pl.pallas_call(
        _tail_body,
        out_shape=jax.ShapeDtypeStruct((ent.shape[0], _HID), jnp.float32),
    )(ent, w1, b1.reshape(1, -1), w2, b2.reshape(1, -1), wp, bp.reshape(1, -1))


def kernel(entity_ids, edge_index, edge_type, bases, comp, root, bias,
           w1, b1, w2, b2, wp, bp):
    src = edge_index[0].astype(jnp.int32)
    dst = edge_index[1].astype(jnp.int32)
    et = edge_type.astype(jnp.int32)
    npad = _EPAD - _NEDGE
    src_p = jnp.concatenate([src, jnp.zeros((npad,), jnp.int32)])
    dst_p = jnp.concatenate(
        [dst, _NE + (jnp.arange(npad, dtype=jnp.int32) % _NSINK)])
    et_p = jnp.concatenate([et, jnp.zeros((npad,), jnp.int32)])

    scales = _sc_hist(dst_p, et_p)
    w = _compute_w(comp, bases)
    w_flat = w.reshape(_NR * _NE, _DE)

    parts = _sc_main(src_p, dst_p, et_p, w_flat, scales)

    root_p = jnp.concatenate(
        [root, jnp.zeros((_OUT_ROWS - _NE, _DE), jnp.float32)])
    ids = entity_ids.reshape(-1).astype(jnp.int32)
    ent = _sc_ent(parts, root_p, bias, ids)
    out_all_p = _sc_sum(parts, root_p, bias)

    entity_embeds_all = out_all_p[:_NE]
    emb = _tc_tail(ent, w1, b1, w2, b2, wp, bp)
    entity_embeds = emb.reshape(entity_ids.shape[0], entity_ids.shape[1], _HID)
    return (entity_embeds, entity_embeds_all)


# speedup vs baseline: 1.0258x; 1.0189x over previous
import functools

import jax
import jax.numpy as jnp
from jax import lax
from jax.experimental import pallas as pl
from jax.experimental.pallas import tpu as pltpu
from jax.experimental.pallas import tpu_sc as plsc

_NE = 10000
_NEDGE = 320000
_NR = 24
_NB = 8
_DE = 128
_HID = 2048

_NC = 2
_NS = 16
_NW = _NC * _NS

_EPAD = 327680
_EPT = _EPAD // _NW
_EPT_H = _EPAD // _NS
_OUT_ROWS = 10240
_RPT = _OUT_ROWS // _NS
_NKEY = 241664
_KPT = _NKEY // _NS
_NSINK = 64
_CB = 2048
_SB = 128
_ZB = _KPT


def _w_body(comp_ref, bases_ref, w_ref):
    b = bases_ref[...].reshape(_NB, -1)
    w = jnp.dot(comp_ref[...], b, preferred_element_type=jnp.float32)
    w_ref[...] = w.reshape(_NR, w_ref.shape[1], _DE)


def _compute_w(comp, bases):
    bi = 1000
    return pl.pallas_call(
        _w_body,
        grid=(_NE // bi,),
        in_specs=[
            pl.BlockSpec((_NR, _NB), lambda i: (0, 0)),
            pl.BlockSpec((_NB, bi, _DE), lambda i: (0, i, 0)),
        ],
        out_specs=pl.BlockSpec((_NR, bi, _DE), lambda i: (0, i, 0)),
        out_shape=jax.ShapeDtypeStruct((_NR, _NE, _DE), jnp.float32),
    )(comp, bases)


def _sc_hist_kernel(dst_h, et_h, scale_h, hist, zb, dstv, etv, keyc, onesb, hsem):
    c = lax.axis_index("c")
    s = lax.axis_index("s")

    def _zb_zero(i, _):
        zb[pl.ds(i * 16, 16)] = jnp.zeros((16,), jnp.float32)
        return 0
    lax.fori_loop(0, _ZB // 16, _zb_zero, 0)
    pltpu.sync_copy(zb, hist.at[pl.ds(s * _KPT, _KPT)])

    def _ones(i, _):
        onesb[pl.ds(i * 16, 16)] = jnp.full((16,), 1.0, jnp.float32)
        return 0
    lax.fori_loop(0, _SB // 16, _ones, 0)

    plsc.subcore_barrier()

    hbase = s * _EPT_H

    def _hchunk(ci, _):
        off = hbase + ci * _CB
        pltpu.sync_copy(dst_h.at[pl.ds(off, _CB)], dstv)
        pltpu.sync_copy(et_h.at[pl.ds(off, _CB)], etv)

        def _hidx(i, _):
            o = pl.multiple_of(i * 16, 16)
            r = i // 8
            q = pl.multiple_of((i % 8) * 16, 16)
            keyc[r, pl.ds(q, 16)] = dstv[pl.ds(o, 16)] * _NR + etv[pl.ds(o, 16)]
            return 0
        lax.fori_loop(0, _CB // 16, _hidx, 0)

        descs = [pltpu.async_copy(onesb, hist.at[keyc.at[j]], hsem, add=True)
                 for j in range(_CB // _SB)]
        for d in descs:
            d.wait()
        return 0
    lax.fori_loop(0, _EPT_H // _CB, _hchunk, 0)

    plsc.subcore_barrier()

    pltpu.sync_copy(hist.at[pl.ds(s * _KPT, _KPT)], zb)

    def _conv(i, _):
        v = zb[pl.ds(i * 16, 16)]
        zb[pl.ds(i * 16, 16)] = 1.0 / jnp.maximum(v, 1.0)
        return 0
    lax.fori_loop(0, _KPT // 16, _conv, 0)

    @pl.when(c == 0)
    def _():
        pltpu.sync_copy(zb, scale_h.at[pl.ds(s * _KPT, _KPT)])


def _sc_hist(dst_p, et_p):
    mesh = plsc.VectorSubcoreMesh(core_axis_name="c", subcore_axis_name="s")
    f = pl.kernel(
        _sc_hist_kernel,
        out_type=jax.ShapeDtypeStruct((_NKEY,), jnp.float32),
        mesh=mesh,
        scratch_types=[
            pltpu.VMEM_SHARED((_NKEY,), jnp.float32),
            pltpu.VMEM((_ZB,), jnp.float32),
            pltpu.VMEM((_CB,), jnp.int32),
            pltpu.VMEM((_CB,), jnp.int32),
            pltpu.VMEM((_CB // _SB, _SB), jnp.int32),
            pltpu.VMEM((_SB,), jnp.float32),
            pltpu.SemaphoreType.DMA,
        ],
    )
    return f(dst_p, et_p)


_NBATCH = _CB // _SB
_CH0 = 7
_CH1 = 3


def _sc_main_kernel(src_h, dst_h, et_h, w_h, scale_h, out_h,
                    acc, srcv, dstv, etv, widxc, keyc, dstc,
                    scaleb0, scaleb1, rowb0, rowb1,
                    gsem0, gsem1, ksem0, ksem1, ssem0, ssem1):
    c = lax.axis_index("c")
    s = lax.axis_index("s")
    rowb = (rowb0, rowb1)
    scaleb = (scaleb0, scaleb1)
    gsem = (gsem0, gsem1)
    ksem = (ksem0, ksem1)
    ssem = (ssem0, ssem1)

    def _rowb_zero(i, _):
        rowb0[i // 8, pl.ds(pl.multiple_of((i % 8) * 16, 16), 16)] = (
            jnp.zeros((16,), jnp.float32))
        return 0
    lax.fori_loop(0, _SB * 8, _rowb_zero, 0)

    def _acc_zero(j, _):
        pltpu.sync_copy(rowb0, acc.at[pl.ds(s * _RPT + j * _SB, _SB)])
        return 0
    lax.fori_loop(0, _RPT // _SB, _acc_zero, 0)

    plsc.subcore_barrier()

    base = s * (_NC * _EPT) + jnp.where(c == 0, 0, _CH0 * _CB)
    nchunks = jnp.where(c == 0, _CH0, _CH1)

    def _chunk(ci, _):
        off = base + ci * _CB
        pltpu.sync_copy(src_h.at[pl.ds(off, _CB)], srcv)
        pltpu.sync_copy(dst_h.at[pl.ds(off, _CB)], dstv)
        pltpu.sync_copy(et_h.at[pl.ds(off, _CB)], etv)

        def _idx(i, _):
            o = pl.multiple_of(i * 16, 16)
            sv = srcv[pl.ds(o, 16)]
            tv = etv[pl.ds(o, 16)]
            dv = dstv[pl.ds(o, 16)]
            r = i // 8
            q = pl.multiple_of((i % 8) * 16, 16)
            widxc[r, pl.ds(q, 16)] = tv * _NE + sv
            keyc[r, pl.ds(q, 16)] = dv * _NR + tv
            dstc[r, pl.ds(q, 16)] = dv
            return 0
        lax.fori_loop(0, _CB // 16, _idx, 0)

        gd = [None, None]
        kd = [None, None]
        sd = [None, None]

        def _issue(k):
            b = k % 2
            if sd[b] is not None:
                sd[b].wait()
            gd[b] = pltpu.async_copy(w_h.at[widxc.at[k]], rowb[b], gsem[b])
            kd[b] = pltpu.async_copy(scale_h.at[keyc.at[k]], scaleb[b], ksem[b])

        _issue(0)
        for j in range(_NBATCH):
            if j + 1 < _NBATCH:
                _issue(j + 1)
            b = j % 2
            gd[b].wait()
            kd[b].wait()

            def _scale(g, _):
                sv = scaleb[b][pl.ds(pl.multiple_of(g * 16, 16), 16)]
                for l in range(16):
                    sc = sv[l]
                    r = g * 16 + l
                    for q in range(_DE // 16):
                        rowb[b][r, pl.ds(q * 16, 16)] = (
                            rowb[b][r, pl.ds(q * 16, 16)] * sc)
                return 0
            lax.fori_loop(0, _SB // 16, _scale, 0)

            sd[b] = pltpu.async_copy(rowb[b], acc.at[dstc.at[j]], ssem[b],
                                     add=True)
        sd[0].wait()
        sd[1].wait()
        return 0
    lax.fori_loop(0, nchunks, _chunk, 0)

    plsc.subcore_barrier()

    row0 = s * _RPT

    def _dump(j, _):
        pltpu.sync_copy(acc.at[pl.ds(row0 + j * _SB, _SB)], rowb0)
        pltpu.sync_copy(rowb0, out_h.at[pl.ds(c * _OUT_ROWS + row0 + j * _SB, _SB)])
        return 0
    lax.fori_loop(0, _RPT // _SB, _dump, 0)


def _sc_main(src_p, dst_p, et_p, w_flat, scales):
    mesh = plsc.VectorSubcoreMesh(core_axis_name="c", subcore_axis_name="s")
    f = pl.kernel(
        _sc_main_kernel,
        out_type=jax.ShapeDtypeStruct((_NC * _OUT_ROWS, _DE), jnp.float32),
        mesh=mesh,
        scratch_types=[
            pltpu.VMEM_SHARED((_OUT_ROWS, _DE), jnp.float32),
            pltpu.VMEM((_CB,), jnp.int32),
            pltpu.VMEM((_CB,), jnp.int32),
            pltpu.VMEM((_CB,), jnp.int32),
            pltpu.VMEM((_NBATCH, _SB), jnp.int32),
            pltpu.VMEM((_NBATCH, _SB), jnp.int32),
            pltpu.VMEM((_NBATCH, _SB), jnp.int32),
            pltpu.VMEM((_SB,), jnp.float32),
            pltpu.VMEM((_SB,), jnp.float32),
            pltpu.VMEM((_SB, _DE), jnp.float32),
            pltpu.VMEM((_SB, _DE), jnp.float32),
            pltpu.SemaphoreType.DMA,
            pltpu.SemaphoreType.DMA,
            pltpu.SemaphoreType.DMA,
            pltpu.SemaphoreType.DMA,
            pltpu.SemaphoreType.DMA,
            pltpu.SemaphoreType.DMA,
        ],
    )
    return f(src_p, dst_p, et_p, w_flat, scales)


_FR = 160


def _sc_sum_kernel(pp, rootp, bias_h, ent_dep, out_all,
                   b0, b1, b2, biasv):
    c = lax.axis_index("c")
    s = lax.axis_index("s")
    gid = s * _NC + c
    pltpu.sync_copy(bias_h, biasv)

    base = gid * (_OUT_ROWS // _NW)

    def _rchunk(j, _):
        r0 = base + j * _FR

        @pl.when(r0 < _NE)
        def _():
            pltpu.sync_copy(pp.at[pl.ds(r0, _FR)], b0)
            pltpu.sync_copy(pp.at[pl.ds(_OUT_ROWS + r0, _FR)], b1)
            pltpu.sync_copy(rootp.at[pl.ds(r0, _FR)], b2)

            def _add(i, _):
                r = i // 8
                q = pl.multiple_of((i % 8) * 16, 16)
                b0[r, pl.ds(q, 16)] = (b0[r, pl.ds(q, 16)] + b1[r, pl.ds(q, 16)]
                                       + b2[r, pl.ds(q, 16)] + biasv[pl.ds(q, 16)])
                return 0
            lax.fori_loop(0, _FR * 8, _add, 0)

            @pl.when(r0 + _FR <= _NE)
            def _():
                pltpu.sync_copy(b0, out_all.at[pl.ds(r0, _FR)])

            @pl.when(r0 + _FR > _NE)
            def _():
                pltpu.sync_copy(b0.at[pl.ds(0, _NE % _FR)],
                                out_all.at[pl.ds(r0, _NE % _FR)])
        return 0
    lax.fori_loop(0, (_OUT_ROWS // _NW) // _FR, _rchunk, 0)


def _sc_sum(parts, root_p, bias, ent_dep):
    mesh = plsc.VectorSubcoreMesh(core_axis_name="c", subcore_axis_name="s")
    f = pl.kernel(
        _sc_sum_kernel,
        out_type=jax.ShapeDtypeStruct((_NE, _DE), jnp.float32),
        mesh=mesh,
        scratch_types=[
            pltpu.VMEM((_FR, _DE), jnp.float32),
            pltpu.VMEM((_FR, _DE), jnp.float32),
            pltpu.VMEM((_FR, _DE), jnp.float32),
            pltpu.VMEM((_DE,), jnp.float32),
        ],
    )
    return f(parts, root_p, bias, ent_dep)


def _sc_ent_kernel(pp, rootp, bias_h, ids_h, ent,
                   biasv, idsb, ids1b, e0, e1, e2):
    c = lax.axis_index("c")
    s = lax.axis_index("s")
    gid = s * _NC + c
    pltpu.sync_copy(bias_h, biasv)

    pltpu.sync_copy(ids_h.at[pl.ds(gid * 32, 32)], idsb)

    def _sh(i, _):
        ids1b[pl.ds(i * 16, 16)] = idsb[pl.ds(i * 16, 16)] + _OUT_ROWS
        return 0
    lax.fori_loop(0, 2, _sh, 0)

    pltpu.sync_copy(pp.at[idsb], e0)
    pltpu.sync_copy(pp.at[ids1b], e1)
    pltpu.sync_copy(rootp.at[idsb], e2)

    def _eadd(i, _):
        r = i // 8
        q = pl.multiple_of((i % 8) * 16, 16)
        e0[r, pl.ds(q, 16)] = (e0[r, pl.ds(q, 16)] + e1[r, pl.ds(q, 16)]
                               + e2[r, pl.ds(q, 16)] + biasv[pl.ds(q, 16)])
        return 0
    lax.fori_loop(0, 32 * 8, _eadd, 0)
    pltpu.sync_copy(e0, ent.at[pl.ds(gid * 32, 32)])


def _sc_ent(parts, root_p, bias, ids):
    mesh = plsc.VectorSubcoreMesh(core_axis_name="c", subcore_axis_name="s")
    f = pl.kernel(
        _sc_ent_kernel,
        out_type=jax.ShapeDtypeStruct((_NW * 32, _DE), jnp.float32),
        mesh=mesh,
        scratch_types=[
            pltpu.VMEM((_DE,), jnp.float32),
            pltpu.VMEM((32,), jnp.int32),
            pltpu.VMEM((32,), jnp.int32),
            pltpu.VMEM((32, _DE), jnp.float32),
            pltpu.VMEM((32, _DE), jnp.float32),
            pltpu.VMEM((32, _DE), jnp.float32),
        ],
    )
    return f(parts, root_p, bias, ids)


def _tail_body(ent_ref, w1_ref, b1_ref, w2_ref, b2_ref, wp_ref, bp_ref, out_ref):
    ent = ent_ref[...]
    h = jnp.maximum(jnp.dot(ent, w1_ref[...], preferred_element_type=jnp.float32)
                    + b1_ref[...], 0.0)
    h = (jnp.dot(h, w2_ref[...], preferred_element_type=jnp.float32)
         + b2_ref[...] + ent)
    out_ref[...] = (jnp.dot(h, wp_ref[...], preferred_element_type=jnp.float32)
                    + bp_ref[...])


def _tc_tail(ent, w1, b1, w2, b2, wp, bp):
    return pl.pallas_call(
        _tail_body,
        out_shape=jax.ShapeDtypeStruct((ent.shape[0], _HID), jnp.float32),
    )(ent, w1, b1.reshape(1, -1), w2, b2.reshape(1, -1), wp, bp.reshape(1, -1))


def kernel(entity_ids, edge_index, edge_type, bases, comp, root, bias,
           w1, b1, w2, b2, wp, bp):
    src = edge_index[0].astype(jnp.int32)
    dst = edge_index[1].astype(jnp.int32)
    et = edge_type.astype(jnp.int32)
    npad = _EPAD - _NEDGE
    src_p = jnp.concatenate([src, jnp.zeros((npad,), jnp.int32)])
    dst_p = jnp.concatenate(
        [dst, _NE + (jnp.arange(npad, dtype=jnp.int32) % _NSINK)])
    et_p = jnp.concatenate([et, jnp.zeros((npad,), jnp.int32)])

    scales = _sc_hist(dst_p, et_p)
    w = _compute_w(comp, bases)
    w_flat = w.reshape(_NR * _NE, _DE)

    parts = _sc_main(src_p, dst_p, et_p, w_flat, scales)

    root_p = jnp.concatenate(
        [root, jnp.zeros((_OUT_ROWS - _NE, _DE), jnp.float32)])
    ids = entity_ids.reshape(-1).astype(jnp.int32)
    ent = _sc_ent(parts, root_p, bias, ids)
    entity_embeds_all = _sc_sum(parts, root_p, bias, ent)

    emb = _tc_tail(ent, w1, b1, w2, b2, wp, bp)
    entity_embeds = emb.reshape(entity_ids.shape[0], entity_ids.shape[1], _HID)
    return (entity_embeds, entity_embeds_all)


# speedup vs baseline: 1.0423x; 1.0162x over previous
import functools

import jax
import jax.numpy as jnp
from jax import lax
from jax.experimental import pallas as pl
from jax.experimental.pallas import tpu as pltpu
from jax.experimental.pallas import tpu_sc as plsc

_NE = 10000
_NEDGE = 320000
_NR = 24
_NB = 8
_DE = 128
_HID = 2048

_NC = 2
_NS = 16
_NW = _NC * _NS

_EPAD = 327680
_EPT = _EPAD // _NW
_EPT_H = _EPAD // _NS
_OUT_ROWS = 10240
_RPT = _OUT_ROWS // _NS
_NKEY = 241664
_KPT = _NKEY // _NS
_NSINK = 64
_CB = 2048
_SB = 128
_ZB = _KPT


def _w_body(comp_ref, bases_ref, w_ref):
    b = bases_ref[...].reshape(_NB, -1)
    w = jnp.dot(comp_ref[...], b, preferred_element_type=jnp.float32)
    w_ref[...] = w.reshape(_NR, w_ref.shape[1], _DE)


def _compute_w(comp, bases):
    bi = 1000
    return pl.pallas_call(
        _w_body,
        grid=(_NE // bi,),
        in_specs=[
            pl.BlockSpec((_NR, _NB), lambda i: (0, 0)),
            pl.BlockSpec((_NB, bi, _DE), lambda i: (0, i, 0)),
        ],
        out_specs=pl.BlockSpec((_NR, bi, _DE), lambda i: (0, i, 0)),
        out_shape=jax.ShapeDtypeStruct((_NR, _NE, _DE), jnp.float32),
    )(comp, bases)


def _sc_hist_kernel(dst_h, et_h, scale_h, hist, zb,
                    dstv0, dstv1, etv0, etv1, keyc0, keyc1, onesb,
                    hsem, lsem0, lsem1):
    c = lax.axis_index("c")
    s = lax.axis_index("s")
    dstv = (dstv0, dstv1)
    etv = (etv0, etv1)
    keyc = (keyc0, keyc1)
    lsem = (lsem0, lsem1)

    def _zb_zero(i, _):
        zb[pl.ds(i * 16, 16)] = jnp.zeros((16,), jnp.float32)
        return 0
    lax.fori_loop(0, _ZB // 16, _zb_zero, 0)
    pltpu.sync_copy(zb, hist.at[pl.ds(s * _KPT, _KPT)])

    def _ones(i, _):
        onesb[pl.ds(i * 16, 16)] = jnp.full((16,), 1.0, jnp.float32)
        return 0
    lax.fori_loop(0, _SB // 16, _ones, 0)

    plsc.subcore_barrier()

    hbase = s * _EPT_H
    nch = _EPT_H // _CB

    def _load(ci, p):
        off = hbase + ci * _CB
        return (pltpu.async_copy(dst_h.at[pl.ds(off, _CB)], dstv[p], lsem[p]),
                pltpu.async_copy(et_h.at[pl.ds(off, _CB)], etv[p], lsem[p]))

    def _keys(p):
        def _hidx(i, _):
            o = pl.multiple_of(i * 16, 16)
            r = i // 8
            q = pl.multiple_of((i % 8) * 16, 16)
            keyc[p][r, pl.ds(q, 16)] = (dstv[p][pl.ds(o, 16)] * _NR
                                        + etv[p][pl.ds(o, 16)])
            return 0
        lax.fori_loop(0, _CB // 16, _hidx, 0)

    d0 = _load(0, 0)
    d0[0].wait()
    d0[1].wait()
    _keys(0)
    for ci in range(nch):
        p = ci % 2
        if ci + 1 < nch:
            dn = _load(ci + 1, 1 - p)
        descs = [pltpu.async_copy(onesb, hist.at[keyc[p].at[j]], hsem, add=True)
                 for j in range(_CB // _SB)]
        if ci + 1 < nch:
            dn[0].wait()
            dn[1].wait()
            _keys(1 - p)
        for d in descs:
            d.wait()

    plsc.subcore_barrier()

    pltpu.sync_copy(hist.at[pl.ds(s * _KPT, _KPT)], zb)

    def _conv(i, _):
        v = zb[pl.ds(i * 16, 16)]
        zb[pl.ds(i * 16, 16)] = 1.0 / jnp.maximum(v, 1.0)
        return 0
    lax.fori_loop(0, _KPT // 16, _conv, 0)

    @pl.when(c == 0)
    def _():
        pltpu.sync_copy(zb, scale_h.at[pl.ds(s * _KPT, _KPT)])


def _sc_hist(dst_p, et_p):
    mesh = plsc.VectorSubcoreMesh(core_axis_name="c", subcore_axis_name="s")
    f = pl.kernel(
        _sc_hist_kernel,
        out_type=jax.ShapeDtypeStruct((_NKEY,), jnp.float32),
        mesh=mesh,
        scratch_types=[
            pltpu.VMEM_SHARED((_NKEY,), jnp.float32),
            pltpu.VMEM((_ZB,), jnp.float32),
            pltpu.VMEM((_CB,), jnp.int32),
            pltpu.VMEM((_CB,), jnp.int32),
            pltpu.VMEM((_CB,), jnp.int32),
            pltpu.VMEM((_CB,), jnp.int32),
            pltpu.VMEM((_CB // _SB, _SB), jnp.int32),
            pltpu.VMEM((_CB // _SB, _SB), jnp.int32),
            pltpu.VMEM((_SB,), jnp.float32),
            pltpu.SemaphoreType.DMA,
            pltpu.SemaphoreType.DMA,
            pltpu.SemaphoreType.DMA,
        ],
    )
    return f(dst_p, et_p)


_NBATCH = _CB // _SB
_CH0 = 7
_CH1 = 3


def _sc_main_kernel(src_h, dst_h, et_h, w_h, scale_h, out_h,
                    acc, srcv, dstv, etv, widxc, keyc, dstc,
                    scaleb0, scaleb1, rowb0, rowb1,
                    gsem0, gsem1, ksem0, ksem1, ssem0, ssem1):
    c = lax.axis_index("c")
    s = lax.axis_index("s")
    rowb = (rowb0, rowb1)
    scaleb = (scaleb0, scaleb1)
    gsem = (gsem0, gsem1)
    ksem = (ksem0, ksem1)
    ssem = (ssem0, ssem1)

    def _rowb_zero(i, _):
        rowb0[i // 8, pl.ds(pl.multiple_of((i % 8) * 16, 16), 16)] = (
            jnp.zeros((16,), jnp.float32))
        return 0
    lax.fori_loop(0, _SB * 8, _rowb_zero, 0)

    def _acc_zero(j, _):
        pltpu.sync_copy(rowb0, acc.at[pl.ds(s * _RPT + j * _SB, _SB)])
        return 0
    lax.fori_loop(0, _RPT // _SB, _acc_zero, 0)

    plsc.subcore_barrier()

    base = s * (_NC * _EPT) + jnp.where(c == 0, 0, _CH0 * _CB)
    nchunks = jnp.where(c == 0, _CH0, _CH1)

    def _chunk(ci, _):
        off = base + ci * _CB
        pltpu.sync_copy(src_h.at[pl.ds(off, _CB)], srcv)
        pltpu.sync_copy(dst_h.at[pl.ds(off, _CB)], dstv)
        pltpu.sync_copy(et_h.at[pl.ds(off, _CB)], etv)

        def _idx(i, _):
            o = pl.multiple_of(i * 16, 16)
            sv = srcv[pl.ds(o, 16)]
            tv = etv[pl.ds(o, 16)]
            dv = dstv[pl.ds(o, 16)]
            r = i // 8
            q = pl.multiple_of((i % 8) * 16, 16)
            widxc[r, pl.ds(q, 16)] = tv * _NE + sv
            keyc[r, pl.ds(q, 16)] = dv * _NR + tv
            dstc[r, pl.ds(q, 16)] = dv
            return 0
        lax.fori_loop(0, _CB // 16, _idx, 0)

        gd = [None, None]
        kd = [None, None]
        sd = [None, None]

        def _issue(k):
            b = k % 2
            if sd[b] is not None:
                sd[b].wait()
            gd[b] = pltpu.async_copy(w_h.at[widxc.at[k]], rowb[b], gsem[b])
            kd[b] = pltpu.async_copy(scale_h.at[keyc.at[k]], scaleb[b], ksem[b])

        _issue(0)
        for j in range(_NBATCH):
            if j + 1 < _NBATCH:
                _issue(j + 1)
            b = j % 2
            gd[b].wait()
            kd[b].wait()

            def _scale(g, _):
                sv = scaleb[b][pl.ds(pl.multiple_of(g * 16, 16), 16)]
                for l in range(16):
                    sc = sv[l]
                    r = g * 16 + l
                    for q in range(_DE // 16):
                        rowb[b][r, pl.ds(q * 16, 16)] = (
                            rowb[b][r, pl.ds(q * 16, 16)] * sc)
                return 0
            lax.fori_loop(0, _SB // 16, _scale, 0)

            sd[b] = pltpu.async_copy(rowb[b], acc.at[dstc.at[j]], ssem[b],
                                     add=True)
        sd[0].wait()
        sd[1].wait()
        return 0
    lax.fori_loop(0, nchunks, _chunk, 0)

    plsc.subcore_barrier()

    row0 = s * _RPT

    def _dump(j, _):
        pltpu.sync_copy(acc.at[pl.ds(row0 + j * _SB, _SB)], rowb0)
        pltpu.sync_copy(rowb0, out_h.at[pl.ds(c * _OUT_ROWS + row0 + j * _SB, _SB)])
        return 0
    lax.fori_loop(0, _RPT // _SB, _dump, 0)


def _sc_main(src_p, dst_p, et_p, w_flat, scales):
    mesh = plsc.VectorSubcoreMesh(core_axis_name="c", subcore_axis_name="s")
    f = pl.kernel(
        _sc_main_kernel,
        out_type=jax.ShapeDtypeStruct((_NC * _OUT_ROWS, _DE), jnp.float32),
        mesh=mesh,
        scratch_types=[
            pltpu.VMEM_SHARED((_OUT_ROWS, _DE), jnp.float32),
            pltpu.VMEM((_CB,), jnp.int32),
            pltpu.VMEM((_CB,), jnp.int32),
            pltpu.VMEM((_CB,), jnp.int32),
            pltpu.VMEM((_NBATCH, _SB), jnp.int32),
            pltpu.VMEM((_NBATCH, _SB), jnp.int32),
            pltpu.VMEM((_NBATCH, _SB), jnp.int32),
            pltpu.VMEM((_SB,), jnp.float32),
            pltpu.VMEM((_SB,), jnp.float32),
            pltpu.VMEM((_SB, _DE), jnp.float32),
            pltpu.VMEM((_SB, _DE), jnp.float32),
            pltpu.SemaphoreType.DMA,
            pltpu.SemaphoreType.DMA,
            pltpu.SemaphoreType.DMA,
            pltpu.SemaphoreType.DMA,
            pltpu.SemaphoreType.DMA,
            pltpu.SemaphoreType.DMA,
        ],
    )
    return f(src_p, dst_p, et_p, w_flat, scales)


_FR = 160


def _sc_sum_kernel(pp, rootp, bias_h, ent_dep, out_all,
                   b0, b1, b2, biasv):
    c = lax.axis_index("c")
    s = lax.axis_index("s")
    gid = s * _NC + c
    pltpu.sync_copy(bias_h, biasv)

    base = gid * (_OUT_ROWS // _NW)

    def _rchunk(j, _):
        r0 = base + j * _FR

        @pl.when(r0 < _NE)
        def _():
            pltpu.sync_copy(pp.at[pl.ds(r0, _FR)], b0)
            pltpu.sync_copy(pp.at[pl.ds(_OUT_ROWS + r0, _FR)], b1)
            pltpu.sync_copy(rootp.at[pl.ds(r0, _FR)], b2)

            def _add(i, _):
                r = i // 8
                q = pl.multiple_of((i % 8) * 16, 16)
                b0[r, pl.ds(q, 16)] = (b0[r, pl.ds(q, 16)] + b1[r, pl.ds(q, 16)]
                                       + b2[r, pl.ds(q, 16)] + biasv[pl.ds(q, 16)])
                return 0
            lax.fori_loop(0, _FR * 8, _add, 0)

            @pl.when(r0 + _FR <= _NE)
            def _():
                pltpu.sync_copy(b0, out_all.at[pl.ds(r0, _FR)])

            @pl.when(r0 + _FR > _NE)
            def _():
                pltpu.sync_copy(b0.at[pl.ds(0, _NE % _FR)],
                                out_all.at[pl.ds(r0, _NE % _FR)])
        return 0
    lax.fori_loop(0, (_OUT_ROWS // _NW) // _FR, _rchunk, 0)


def _sc_sum(parts, root_p, bias, ent_dep):
    mesh = plsc.VectorSubcoreMesh(core_axis_name="c", subcore_axis_name="s")
    f = pl.kernel(
        _sc_sum_kernel,
        out_type=jax.ShapeDtypeStruct((_NE, _DE), jnp.float32),
        mesh=mesh,
        scratch_types=[
            pltpu.VMEM((_FR, _DE), jnp.float32),
            pltpu.VMEM((_FR, _DE), jnp.float32),
            pltpu.VMEM((_FR, _DE), jnp.float32),
            pltpu.VMEM((_DE,), jnp.float32),
        ],
    )
    return f(parts, root_p, bias, ent_dep)


def _sc_ent_kernel(pp, rootp, bias_h, ids_h, ent,
                   biasv, idsb, ids1b, e0, e1, e2):
    c = lax.axis_index("c")
    s = lax.axis_index("s")
    gid = s * _NC + c
    pltpu.sync_copy(bias_h, biasv)

    pltpu.sync_copy(ids_h.at[pl.ds(gid * 32, 32)], idsb)

    def _sh(i, _):
        ids1b[pl.ds(i * 16, 16)] = idsb[pl.ds(i * 16, 16)] + _OUT_ROWS
        return 0
    lax.fori_loop(0, 2, _sh, 0)

    pltpu.sync_copy(pp.at[idsb], e0)
    pltpu.sync_copy(pp.at[ids1b], e1)
    pltpu.sync_copy(rootp.at[idsb], e2)

    def _eadd(i, _):
        r = i // 8
        q = pl.multiple_of((i % 8) * 16, 16)
        e0[r, pl.ds(q, 16)] = (e0[r, pl.ds(q, 16)] + e1[r, pl.ds(q, 16)]
                               + e2[r, pl.ds(q, 16)] + biasv[pl.ds(q, 16)])
        return 0
    lax.fori_loop(0, 32 * 8, _eadd, 0)
    pltpu.sync_copy(e0, ent.at[pl.ds(gid * 32, 32)])


def _sc_ent(parts, root_p, bias, ids):
    mesh = plsc.VectorSubcoreMesh(core_axis_name="c", subcore_axis_name="s")
    f = pl.kernel(
        _sc_ent_kernel,
        out_type=jax.ShapeDtypeStruct((_NW * 32, _DE), jnp.float32),
        mesh=mesh,
        scratch_types=[
            pltpu.VMEM((_DE,), jnp.float32),
            pltpu.VMEM((32,), jnp.int32),
            pltpu.VMEM((32,), jnp.int32),
            pltpu.VMEM((32, _DE), jnp.float32),
            pltpu.VMEM((32, _DE), jnp.float32),
            pltpu.VMEM((32, _DE), jnp.float32),
        ],
    )
    return f(parts, root_p, bias, ids)


def _tail_body(ent_ref, w1_ref, b1_ref, w2_ref, b2_ref, wp_ref, bp_ref, out_ref):
    ent = ent_ref[...]
    h = jnp.maximum(jnp.dot(ent, w1_ref[...], preferred_element_type=jnp.float32)
                    + b1_ref[...], 0.0)
    h = (jnp.dot(h, w2_ref[...], preferred_element_type=jnp.float32)
         + b2_ref[...] + ent)
    out_ref[...] = (jnp.dot(h, wp_ref[...], preferred_element_type=jnp.float32)
                    + bp_ref[...])


def _tc_tail(ent, w1, b1, w2, b2, wp, bp):
    return pl.pallas_call(
        _tail_body,
        out_shape=jax.ShapeDtypeStruct((ent.shape[0], _HID), jnp.float32),
    )(ent, w1, b1.reshape(1, -1), w2, b2.reshape(1, -1), wp, bp.reshape(1, -1))


def kernel(entity_ids, edge_index, edge_type, bases, comp, root, bias,
           w1, b1, w2, b2, wp, bp):
    src = edge_index[0].astype(jnp.int32)
    dst = edge_index[1].astype(jnp.int32)
    et = edge_type.astype(jnp.int32)
    npad = _EPAD - _NEDGE
    src_p = jnp.concatenate([src, jnp.zeros((npad,), jnp.int32)])
    dst_p = jnp.concatenate(
        [dst, _NE + (jnp.arange(npad, dtype=jnp.int32) % _NSINK)])
    et_p = jnp.concatenate([et, jnp.zeros((npad,), jnp.int32)])

    scales = _sc_hist(dst_p, et_p)
    w = _compute_w(comp, bases)
    w_flat = w.reshape(_NR * _NE, _DE)

    parts = _sc_main(src_p, dst_p, et_p, w_flat, scales)

    root_p = jnp.concatenate(
        [root, jnp.zeros((_OUT_ROWS - _NE, _DE), jnp.float32)])
    ids = entity_ids.reshape(-1).astype(jnp.int32)
    ent = _sc_ent(parts, root_p, bias, ids)
    entity_embeds_all = _sc_sum(parts, root_p, bias, ent)

    emb = _tc_tail(ent, w1, b1, w2, b2, wp, bp)
    entity_embeds = emb.reshape(entity_ids.shape[0], entity_ids.shape[1], _HID)
    return (entity_embeds, entity_embeds_all)
